# Initial kernel scaffold; baseline (speedup 1.0000x reference)
import jax
import jax.numpy as jnp
from jax.experimental import pallas as pl


def kernel(x):
    raise NotImplementedError("write your pallas kernel here")


# speedup vs baseline: 13.4680x; 13.4680x over previous
import functools

import jax
import jax.numpy as jnp
from jax.experimental import pallas as pl
from jax.experimental.pallas import tpu as pltpu

_K = 15000
_ROWS = 2352
_LANES = 128


def _order_i32(s):
    return s ^ ((s >> 31) & jnp.int32(0x7FFFFFFF))


def _tc_body(x_ref, o_ref, ord_scratch):
    x = x_ref[0]
    s = jax.lax.bitcast_convert_type(x, jnp.int32)
    ordx = _order_i32(s)
    ord_scratch[...] = ordx

    c_nonneg = jnp.sum((ordx >= 0).astype(jnp.int32))
    take_pos = c_nonneg >= _K
    lo0 = jnp.where(take_pos, jnp.int32(0), jnp.int32(-2147483648))
    hi0 = jnp.where(take_pos, jnp.int32(2147483647), jnp.int32(-1))

    def body(_, carry):
        lo, hi = carry
        span = hi - lo
        mid = lo + (span >> 1) + (span & 1)
        cnt = jnp.sum((ord_scratch[...] >= mid).astype(jnp.int32))
        ge = cnt >= _K
        return jnp.where(ge, mid, lo), jnp.where(ge, hi, mid - 1)

    lo, _ = jax.lax.fori_loop(0, 31, body, (lo0, hi0))
    tbits = _order_i32(lo)
    t = jax.lax.bitcast_convert_type(tbits, jnp.float32)
    o_ref[0] = jnp.where(x >= t, x, jnp.float32(0.0))


@jax.jit
def kernel(x):
    B = x.shape[0]
    x2 = x.reshape(B, _ROWS, _LANES)
    out = pl.pallas_call(
        _tc_body,
        grid=(B,),
        in_specs=[pl.BlockSpec((1, _ROWS, _LANES), lambda i: (i, 0, 0))],
        out_specs=pl.BlockSpec((1, _ROWS, _LANES), lambda i: (i, 0, 0)),
        out_shape=jax.ShapeDtypeStruct((B, _ROWS, _LANES), jnp.float32),
        scratch_shapes=[pltpu.VMEM((_ROWS, _LANES), jnp.int32)],
    )(x2)
    return out.reshape(x.shape)
